# baseline (speedup 1.0000x reference)
import numpy as np
import jax
import jax.numpy as jnp
from jax.experimental import pallas as pl
from jax.experimental.pallas import tpu as pltpu

_jnp_array_orig = jnp.array


def _jnp_array_host_complex(obj, dtype=None, **kw):
    try:
        wants_complex = (
            dtype is not None
            and jnp.issubdtype(jnp.dtype(dtype), jnp.complexfloating)
            and isinstance(obj, (list, tuple, np.ndarray))
        )
    except Exception:
        wants_complex = False
    if wants_complex:
        return np.array(obj, dtype=np.dtype(dtype))
    return _jnp_array_orig(obj, dtype=dtype, **kw)


jnp.array = _jnp_array_host_complex

N_QUBITS = 16
N_LAYERS = 6
DIM = 1 << N_QUBITS
N_SAMPLES = 16384
SB = 32
N_BLOCKS = N_SAMPLES // SB
CHUNK = 256


def _bitperm_matrix(perm):
    m = np.zeros((256, 256), np.float32)
    for x in range(256):
        m[perm(x), x] = 1.0
    return m


def _build_consts():
    xs = []
    for qq in range(8):
        mask = 1 << (7 - qq)
        xs.append(_bitperm_matrix(lambda x, m=mask: x ^ m))
    prow = []
    for qq in range(7):
        cm, tm = 1 << (7 - qq), 1 << (7 - (qq + 1))

        def f(x, cm=cm, tm=tm):
            return x ^ (tm if x & cm else 0)
        prow.append(_bitperm_matrix(f))
    pcol = []
    for jj in range(7):
        cm, tm = 1 << (7 - jj), 1 << (7 - (jj + 1))

        def f(x, cm=cm, tm=tm):
            return x ^ (tm if x & cm else 0)
        m = np.zeros((256, 256), np.float32)
        for x in range(256):
            m[f(x), x] = 1.0
        pcol.append(m)
    qm = np.zeros((256, 256), np.float32)
    for c in range(256):
        qm[c ^ 128, c] = 1.0
    return xs, prow, pcol, qm


_XS, _PROW, _PCOL, _QM = _build_consts()
_CONSTS = _XS + _PROW + _PCOL + [_QM]

_DOT = dict(preferred_element_type=jnp.float32)


def _bq(x):
    return x.astype(jnp.bfloat16).astype(jnp.float32)


def _bq_bits(x):
    u = jax.lax.bitcast_convert_type(x, jnp.uint32)
    r = u + np.uint32(0x7FFF) + ((u >> np.uint32(16)) & np.uint32(1))
    return jax.lax.bitcast_convert_type(r & np.uint32(0xFFFF0000), jnp.float32)


def _probs_kernel(coef_ref, *refs):
    xs = refs[0:8]
    prow = refs[8:15]
    pcol = refs[15:22]
    qm = refs[22]
    out_ref = refs[23]

    ri = jax.lax.broadcasted_iota(jnp.int32, (256, 256), 0)
    ci = jax.lax.broadcasted_iota(jnp.int32, (256, 256), 1)
    rowbit = [((ri >> (7 - qq)) & 1) == 1 for qq in range(8)]
    colbit = [((ci >> (7 - qq)) & 1) == 1 for qq in range(8)]
    rowodd = rowbit[7]

    s_re = jnp.where((ri == 0) & (ci == 0), 1.0, 0.0).astype(jnp.float32)
    s_im = jnp.zeros((256, 256), jnp.float32)

    def lmul(m, x):
        return jnp.dot(m[...], x, **_DOT)

    def rmul(x, m):
        return jnp.dot(x, m[...], **_DOT)

    def apply_rx(s_re, s_im, qq, qc, qs):
        row = qq < 8
        x = xs[qq if row else qq - 8]
        qre = _bq_bits(s_re)
        d2 = _bq_bits(s_im - s_re)
        s2 = _bq_bits(s_re + s_im)
        t1 = lmul(x, qre) if row else rmul(qre, x)
        t2 = lmul(x, s2) if row else rmul(s2, x)
        k1 = qc * qre - qs * t1
        k2 = qc * d2
        k3 = -qs * t2
        return k1 - k3, k1 + k2

    def apply_rz(s_re, s_im, qq, a0, a1, b0, b1, c0, c1):
        bit = rowbit[qq] if qq < 8 else colbit[qq - 8]
        co1 = jnp.where(bit, a1, a0)
        co2 = jnp.where(bit, b1, b0)
        co3 = jnp.where(bit, c1, c0)
        qre = _bq_bits(s_re)
        d2 = _bq_bits(s_im - s_re)
        s2 = _bq_bits(s_re + s_im)
        k1 = co1 * qre
        k2 = co2 * d2
        k3 = co3 * s2
        return k1 - k3, k1 + k2

    def apply_perm_cnot(s_re, s_im, perm_apply):
        qre = _bq_bits(s_re)
        d = _bq_bits(s_im - s_re)
        pre = perm_apply(qre)
        return pre, pre + perm_apply(d)

    for l in range(N_LAYERS):
        for qq in range(16):
            base = 8 * qq
            qc = coef_ref[l, base + 0]
            qs = coef_ref[l, base + 1]
            s_re, s_im = apply_rx(s_re, s_im, qq, qc, qs)
            a0 = coef_ref[l, base + 2]
            a1 = coef_ref[l, base + 3]
            b0 = coef_ref[l, base + 4]
            b1 = coef_ref[l, base + 5]
            c0 = coef_ref[l, base + 6]
            c1 = coef_ref[l, base + 7]
            s_re, s_im = apply_rz(s_re, s_im, qq, a0, a1, b0, b1, c0, c1)
        if l < N_LAYERS - 1:
            for qq in range(7):
                s_re, s_im = apply_perm_cnot(
                    s_re, s_im, lambda x, m=prow[qq]: lmul(m, x))
            s_re, s_im = apply_perm_cnot(
                s_re, s_im,
                lambda x: jnp.where(rowodd, rmul(x, qm), x))
            for jj in range(7):
                s_re, s_im = apply_perm_cnot(
                    s_re, s_im, lambda x, m=pcol[jj]: rmul(x, m))

    p = s_re * s_re + s_im * s_im
    out_ref[...] = np.float32(0.69314718056) / (p + 1e-12)


_KS0 = np.uint32(0)
_KS1 = np.uint32(42)
_KS2 = np.uint32(0x1BD11BDA) ^ _KS1
_ROTS_A = (13, 15, 26, 6)
_ROTS_B = (17, 29, 16, 24)


def _tf_rounds(x0, x1, rots):
    for r in rots:
        x0 = x0 + x1
        x1 = (x1 << np.uint32(r)) | (x1 >> np.uint32(32 - r))
        x1 = x1 ^ x0
    return x0, x1


def _threefry_xor(cnt):
    x0 = jnp.zeros_like(cnt) + _KS0
    x1 = cnt + _KS1
    ks = (_KS0, _KS1, _KS2)
    for i in range(5):
        x0, x1 = _tf_rounds(x0, x1, _ROTS_A if i % 2 == 0 else _ROTS_B)
        x0 = x0 + ks[(i + 1) % 3]
        x1 = x1 + ks[(i + 2) % 3] + np.uint32(i + 1)
    return x0 ^ x1


def _sample_kernel(rr_ref, out_ref):
    g = pl.program_id(0)
    srow = jax.lax.broadcasted_iota(jnp.uint32, (SB, CHUNK), 0)
    lane = jax.lax.broadcasted_iota(jnp.uint32, (SB, CHUNK), 1)
    base = (g.astype(jnp.uint32) * np.uint32(SB) + srow) * np.uint32(DIM) + lane

    tiny = np.float32(np.finfo(np.float32).tiny)
    neg_inf = np.float32(-3.4e38)

    def body(k, carry):
        acc_v, acc_k = carry
        cnt = base + (k.astype(jnp.uint32) << np.uint32(8))
        bits = _threefry_xor(cnt)
        fb = (bits >> np.uint32(9)) | np.uint32(0x3F800000)
        f = jax.lax.bitcast_convert_type(fb, jnp.float32) - np.float32(1.0)
        u = jnp.maximum(f, tiny)
        v = jnp.log2(u) * rr_ref[pl.ds(k, 1), :]
        upd = v > acc_v
        acc_v = jnp.where(upd, v, acc_v)
        acc_k = jnp.where(upd, k, acc_k)
        return acc_v, acc_k

    acc_v0 = jnp.full((SB, CHUNK), neg_inf, jnp.float32)
    acc_k0 = jnp.zeros((SB, CHUNK), jnp.int32)
    acc_v, acc_k = jax.lax.fori_loop(0, 256, body, (acc_v0, acc_k0))

    m = jnp.max(acc_v, axis=1, keepdims=True)
    lane_i = jax.lax.broadcasted_iota(jnp.int32, (SB, CHUNK), 1)
    cand = jnp.where(acc_v >= m, acc_k * 256 + lane_i, jnp.int32(2 ** 30))
    idx = jnp.min(cand, axis=1, keepdims=True)

    shifts = (N_QUBITS - 1) - jax.lax.broadcasted_iota(jnp.int32, (SB, N_QUBITS), 1)
    out_ref[...] = ((idx >> shifts) & 1).astype(jnp.float32)


def _coef_table(params):
    th = params[:N_LAYERS * 32].reshape(N_LAYERS, 16, 2)
    thx, thz = th[..., 0], th[..., 1]
    cx = jnp.cos(thx / 2)
    sx = jnp.sin(thx / 2)
    e0 = jnp.exp(-1j * thz.astype(jnp.complex64) / 2)
    e1 = jnp.exp(1j * thz.astype(jnp.complex64) / 2)
    czm, szm = jnp.real(e0), jnp.imag(e0)
    czp, szp = jnp.real(e1), jnp.imag(e1)
    cols = [
        _bq(cx), _bq(sx),
        _bq(czm + szm), _bq(czp + szp),
        _bq(czm), _bq(czp),
        _bq(szm), _bq(szp),
    ]
    coef = jnp.stack(cols, axis=-1)
    return coef.reshape(N_LAYERS, 128)


def kernel(params, n_samples):
    del n_samples
    params = params.astype(jnp.float32)
    coef = _coef_table(params)

    consts = [jnp.asarray(c) for c in _CONSTS]
    rr = pl.pallas_call(
        _probs_kernel,
        out_shape=jax.ShapeDtypeStruct((256, 256), jnp.float32),
        in_specs=[pl.BlockSpec(memory_space=pltpu.SMEM)]
        + [pl.BlockSpec(memory_space=pltpu.VMEM)] * 23,
        out_specs=pl.BlockSpec(memory_space=pltpu.VMEM),
    )(coef, *consts)

    bits = pl.pallas_call(
        _sample_kernel,
        grid=(N_BLOCKS,),
        out_shape=jax.ShapeDtypeStruct((N_SAMPLES, N_QUBITS), jnp.float32),
        in_specs=[pl.BlockSpec((256, 256), lambda g: (0, 0))],
        out_specs=pl.BlockSpec((SB, N_QUBITS), lambda g: (g, 0)),
    )(rr)
    return bits


# speedup vs baseline: 1.0073x; 1.0073x over previous
import numpy as np
import jax
import jax.numpy as jnp
from jax.experimental import pallas as pl
from jax.experimental.pallas import tpu as pltpu

_jnp_array_orig = jnp.array


def _jnp_array_host_complex(obj, dtype=None, **kw):
    try:
        wants_complex = (
            dtype is not None
            and jnp.issubdtype(jnp.dtype(dtype), jnp.complexfloating)
            and isinstance(obj, (list, tuple, np.ndarray))
        )
    except Exception:
        wants_complex = False
    if wants_complex:
        return np.array(obj, dtype=np.dtype(dtype))
    return _jnp_array_orig(obj, dtype=dtype, **kw)


jnp.array = _jnp_array_host_complex

N_QUBITS = 16
N_LAYERS = 6
DIM = 1 << N_QUBITS
N_SAMPLES = 16384
SB = 32
N_BLOCKS = N_SAMPLES // SB
CHUNK = 256


def _bitperm_matrix(perm):
    m = np.zeros((256, 256), np.float32)
    for x in range(256):
        m[perm(x), x] = 1.0
    return m


def _build_consts():
    xs = []
    for qq in range(8):
        mask = 1 << (7 - qq)
        xs.append(_bitperm_matrix(lambda x, m=mask: x ^ m))
    prow = []
    for qq in range(7):
        cm, tm = 1 << (7 - qq), 1 << (7 - (qq + 1))

        def f(x, cm=cm, tm=tm):
            return x ^ (tm if x & cm else 0)
        prow.append(_bitperm_matrix(f))
    pcol = []
    for jj in range(7):
        cm, tm = 1 << (7 - jj), 1 << (7 - (jj + 1))

        def f(x, cm=cm, tm=tm):
            return x ^ (tm if x & cm else 0)
        m = np.zeros((256, 256), np.float32)
        for x in range(256):
            m[f(x), x] = 1.0
        pcol.append(m)
    qm = np.zeros((256, 256), np.float32)
    for c in range(256):
        qm[c ^ 128, c] = 1.0
    return xs, prow, pcol, qm


_XS, _PROW, _PCOL, _QM = _build_consts()
_CONSTS = _XS + _PROW + _PCOL + [_QM]

_DOT = dict(preferred_element_type=jnp.float32)


def _bq(x):
    return x.astype(jnp.bfloat16).astype(jnp.float32)


def _bq_bits(x):
    u = jax.lax.bitcast_convert_type(x, jnp.uint32)
    r = u + np.uint32(0x7FFF) + ((u >> np.uint32(16)) & np.uint32(1))
    return jax.lax.bitcast_convert_type(r & np.uint32(0xFFFF0000), jnp.float32)


def _probs_kernel(coef_ref, *refs):
    xs = refs[0:8]
    prow = refs[8:15]
    pcol = refs[15:22]
    qm = refs[22]
    out_ref = refs[23]

    ri = jax.lax.broadcasted_iota(jnp.int32, (256, 256), 0)
    ci = jax.lax.broadcasted_iota(jnp.int32, (256, 256), 1)
    rowbit = [((ri >> (7 - qq)) & 1) == 1 for qq in range(8)]
    colbit = [((ci >> (7 - qq)) & 1) == 1 for qq in range(8)]
    rowodd = rowbit[7]

    s_re = jnp.where((ri == 0) & (ci == 0), 1.0, 0.0).astype(jnp.float32)
    s_im = jnp.zeros((256, 256), jnp.float32)

    def lmul(m, x):
        return jnp.dot(m[...], x, **_DOT)

    def rmul(x, m):
        return jnp.dot(x, m[...], **_DOT)

    def apply_rx(s_re, s_im, qq, qc, qs):
        row = qq < 8
        x = xs[qq if row else qq - 8]
        qre = _bq_bits(s_re)
        d2 = _bq_bits(s_im - s_re)
        s2 = _bq_bits(s_re + s_im)
        t1 = lmul(x, qre) if row else rmul(qre, x)
        t2 = lmul(x, s2) if row else rmul(s2, x)
        k1 = qc * qre - qs * t1
        k2 = qc * d2
        k3 = -qs * t2
        return k1 - k3, k1 + k2

    def apply_rz(s_re, s_im, qq, a0, a1, b0, b1, c0, c1):
        bit = rowbit[qq] if qq < 8 else colbit[qq - 8]
        co1 = jnp.where(bit, a1, a0)
        co2 = jnp.where(bit, b1, b0)
        co3 = jnp.where(bit, c1, c0)
        qre = _bq_bits(s_re)
        d2 = _bq_bits(s_im - s_re)
        s2 = _bq_bits(s_re + s_im)
        k1 = co1 * qre
        k2 = co2 * d2
        k3 = co3 * s2
        return k1 - k3, k1 + k2

    def apply_perm_cnot(s_re, s_im, perm_apply):
        qre = _bq_bits(s_re)
        d = _bq_bits(s_im - s_re)
        pre = perm_apply(qre)
        return pre, pre + perm_apply(d)

    for l in range(N_LAYERS):
        for qq in range(16):
            base = 8 * qq
            qc = coef_ref[l, base + 0]
            qs = coef_ref[l, base + 1]
            s_re, s_im = apply_rx(s_re, s_im, qq, qc, qs)
            a0 = coef_ref[l, base + 2]
            a1 = coef_ref[l, base + 3]
            b0 = coef_ref[l, base + 4]
            b1 = coef_ref[l, base + 5]
            c0 = coef_ref[l, base + 6]
            c1 = coef_ref[l, base + 7]
            s_re, s_im = apply_rz(s_re, s_im, qq, a0, a1, b0, b1, c0, c1)
        if l < N_LAYERS - 1:
            for qq in range(7):
                s_re, s_im = apply_perm_cnot(
                    s_re, s_im, lambda x, m=prow[qq]: lmul(m, x))
            s_re, s_im = apply_perm_cnot(
                s_re, s_im,
                lambda x: jnp.where(rowodd, rmul(x, qm), x))
            for jj in range(7):
                s_re, s_im = apply_perm_cnot(
                    s_re, s_im, lambda x, m=pcol[jj]: rmul(x, m))

    p = s_re * s_re + s_im * s_im
    out_ref[...] = 1.0 / (p + 1e-12)


_KS0 = np.uint32(0)
_KS1 = np.uint32(42)
_KS2 = np.uint32(0x1BD11BDA) ^ _KS1
_ROTS_A = (13, 15, 26, 6)
_ROTS_B = (17, 29, 16, 24)


def _tf_rounds(x0, x1, rots):
    for r in rots:
        x0 = x0 + x1
        x1 = (x1 << np.uint32(r)) | (x1 >> np.uint32(32 - r))
        x1 = x1 ^ x0
    return x0, x1


def _threefry_xor(cnt):
    x0 = jnp.zeros_like(cnt) + _KS0
    x1 = cnt + _KS1
    ks = (_KS0, _KS1, _KS2)
    for i in range(5):
        x0, x1 = _tf_rounds(x0, x1, _ROTS_A if i % 2 == 0 else _ROTS_B)
        x0 = x0 + ks[(i + 1) % 3]
        x1 = x1 + ks[(i + 2) % 3] + np.uint32(i + 1)
    return x0 ^ x1


def _sample_kernel(rr_ref, out_ref):
    g = pl.program_id(0)
    srow = jax.lax.broadcasted_iota(jnp.uint32, (SB, CHUNK), 0)
    lane = jax.lax.broadcasted_iota(jnp.uint32, (SB, CHUNK), 1)
    base = (g.astype(jnp.uint32) * np.uint32(SB) + srow) * np.uint32(DIM) + lane

    tiny = np.float32(np.finfo(np.float32).tiny)
    neg_inf = np.float32(-3.4e38)

    def body(k, carry):
        acc_v, acc_k = carry
        cnt = base + (k.astype(jnp.uint32) << np.uint32(8))
        bits = _threefry_xor(cnt)
        fb = (bits >> np.uint32(9)) | np.uint32(0x3F800000)
        f = jax.lax.bitcast_convert_type(fb, jnp.float32) - np.float32(1.0)
        u = jnp.maximum(f, tiny)
        v = jnp.log(u) * rr_ref[pl.ds(k, 1), :]
        upd = v > acc_v
        acc_v = jnp.where(upd, v, acc_v)
        acc_k = jnp.where(upd, k, acc_k)
        return acc_v, acc_k

    acc_v0 = jnp.full((SB, CHUNK), neg_inf, jnp.float32)
    acc_k0 = jnp.zeros((SB, CHUNK), jnp.int32)
    acc_v, acc_k = jax.lax.fori_loop(0, 256, body, (acc_v0, acc_k0))

    m = jnp.max(acc_v, axis=1, keepdims=True)
    lane_i = jax.lax.broadcasted_iota(jnp.int32, (SB, CHUNK), 1)
    cand = jnp.where(acc_v >= m, acc_k * 256 + lane_i, jnp.int32(2 ** 30))
    idx = jnp.min(cand, axis=1, keepdims=True)

    shifts = (N_QUBITS - 1) - jax.lax.broadcasted_iota(jnp.int32, (SB, N_QUBITS), 1)
    out_ref[...] = ((idx >> shifts) & 1).astype(jnp.float32)


def _coef_table(params):
    th = params[:N_LAYERS * 32].reshape(N_LAYERS, 16, 2)
    thx, thz = th[..., 0], th[..., 1]
    cx = jnp.cos(thx / 2)
    sx = jnp.sin(thx / 2)
    e0 = jnp.exp(-1j * thz.astype(jnp.complex64) / 2)
    e1 = jnp.exp(1j * thz.astype(jnp.complex64) / 2)
    czm, szm = jnp.real(e0), jnp.imag(e0)
    czp, szp = jnp.real(e1), jnp.imag(e1)
    cols = [
        _bq(cx), _bq(sx),
        _bq(czm + szm), _bq(czp + szp),
        _bq(czm), _bq(czp),
        _bq(szm), _bq(szp),
    ]
    coef = jnp.stack(cols, axis=-1)
    return coef.reshape(N_LAYERS, 128)


def kernel(params, n_samples):
    del n_samples
    params = params.astype(jnp.float32)
    coef = _coef_table(params)

    consts = [jnp.asarray(c) for c in _CONSTS]
    rr = pl.pallas_call(
        _probs_kernel,
        out_shape=jax.ShapeDtypeStruct((256, 256), jnp.float32),
        in_specs=[pl.BlockSpec(memory_space=pltpu.SMEM)]
        + [pl.BlockSpec(memory_space=pltpu.VMEM)] * 23,
        out_specs=pl.BlockSpec(memory_space=pltpu.VMEM),
    )(coef, *consts)

    bits = pl.pallas_call(
        _sample_kernel,
        grid=(N_BLOCKS,),
        out_shape=jax.ShapeDtypeStruct((N_SAMPLES, N_QUBITS), jnp.float32),
        in_specs=[pl.BlockSpec((256, 256), lambda g: (0, 0))],
        out_specs=pl.BlockSpec((SB, N_QUBITS), lambda g: (g, 0)),
    )(rr)
    return bits
